# baseline (speedup 1.0000x reference)
import functools

import jax
import jax.numpy as jnp
from jax import lax
from jax.experimental import pallas as pl
from jax.experimental.pallas import tpu as pltpu
from jax.experimental.pallas import tpu_sc as plsc

_HIDDEN = 2048
_BATCH = 16384
_NC = 2
_NS = 16
_R0 = 576
_R1 = _BATCH // _NS - _R0

_mesh = plsc.VectorSubcoreMesh(core_axis_name="c", subcore_axis_name="s")


@functools.partial(
    pl.kernel,
    out_type=jax.ShapeDtypeStruct((_BATCH, _HIDDEN), jnp.float32),
    mesh=_mesh,
    scratch_types=[
        pltpu.VMEM((1, _HIDDEN), jnp.float32),
        pltpu.SemaphoreType.DMA,
    ],
)
def _broadcast_row(w_hbm, out_hbm, buf, sem):
    c = lax.axis_index("c")
    s = lax.axis_index("s")
    pltpu.sync_copy(w_hbm, buf)

    @pl.when(c == 0)
    def _():
        base = s * _R0
        copies = [
            pltpu.async_copy(buf, out_hbm.at[pl.ds(base + i, 1)], sem)
            for i in range(_R0)
        ]
        for cp in copies:
            cp.wait()

    @pl.when(c == 1)
    def _():
        base = _NS * _R0 + s * _R1
        copies = [
            pltpu.async_copy(buf, out_hbm.at[pl.ds(base + i, 1)], sem)
            for i in range(_R1)
        ]
        for cp in copies:
            cp.wait()


def kernel(x, embedding_weight):
    del x
    return _broadcast_row(embedding_weight)


# speedup vs baseline: 1.0301x; 1.0301x over previous
import functools

import jax
import jax.numpy as jnp
from jax import lax
from jax.experimental import pallas as pl
from jax.experimental.pallas import tpu as pltpu
from jax.experimental.pallas import tpu_sc as plsc

_HIDDEN = 2048
_BATCH = 16384
_NC = 2
_NS = 16
_R0 = 448
_R1 = _BATCH // _NS - _R0

_mesh = plsc.VectorSubcoreMesh(core_axis_name="c", subcore_axis_name="s")


@functools.partial(
    pl.kernel,
    out_type=jax.ShapeDtypeStruct((_BATCH, _HIDDEN), jnp.float32),
    mesh=_mesh,
    scratch_types=[
        pltpu.VMEM((1, _HIDDEN), jnp.float32),
        pltpu.SemaphoreType.DMA,
    ],
)
def _broadcast_row(w_hbm, out_hbm, buf, sem):
    c = lax.axis_index("c")
    s = lax.axis_index("s")
    pltpu.sync_copy(w_hbm, buf)

    @pl.when(c == 0)
    def _():
        base = s * _R0
        copies = [
            pltpu.async_copy(buf, out_hbm.at[pl.ds(base + i, 1)], sem)
            for i in range(_R0)
        ]
        for cp in copies:
            cp.wait()

    @pl.when(c == 1)
    def _():
        base = _NS * _R0 + s * _R1
        copies = [
            pltpu.async_copy(buf, out_hbm.at[pl.ds(base + i, 1)], sem)
            for i in range(_R1)
        ]
        for cp in copies:
            cp.wait()


def kernel(x, embedding_weight):
    del x
    return _broadcast_row(embedding_weight)
